# baseline (speedup 1.0000x reference)
import functools

import jax
import jax.numpy as jnp
from jax import lax
from jax.experimental import pallas as pl
from jax.experimental.pallas import tpu as pltpu
from jax.experimental.pallas import tpu_sc as plsc

EMB = 64
NC, NS = 2, 16
NW = NC * NS
TB = 128
NJ = 200
NBUF = 8
NOB = 2
LANES = 16


@functools.lru_cache(maxsize=None)
def _build_gather():
    mesh = plsc.VectorSubcoreMesh(core_axis_name="c", subcore_axis_name="s")

    def body(idx_hbm, table_hbm, out_hbm, idxblk, hidx, *scratch):
        gbufs = scratch[:NBUF]
        obufs = scratch[NBUF:NBUF + NOB]
        gsems = scratch[NBUF + NOB:2 * NBUF + NOB]
        wsems = scratch[2 * NBUF + NOB:]
        wid = lax.axis_index("s") * NC + lax.axis_index("c")

        pltpu.sync_copy(idx_hbm.at[wid], idxblk)

        iota = lax.iota(jnp.int32, LANES)
        cvecs = [iota + LANES * cb for cb in range(TB // LANES)]

        def build_hidx(j, b):
            jv = jnp.full((LANES,), j, dtype=jnp.int32)
            for cb in range(TB // LANES):
                v = plsc.load_gather(idxblk, [cvecs[cb], jv])
                hidx[b, pl.ds(LANES * cb, LANES)] = v

        def start_gather(b):
            pltpu.async_copy(table_hbm.at[hidx.at[b]], gbufs[b], gsems[b])

        def wait_gather(b):
            pltpu.make_async_copy(
                table_hbm.at[hidx.at[b]], gbufs[b], gsems[b]).wait()

        Rv = [(iota + u0) // 8 for u0 in range(0, EMB, LANES)]
        rv = [(iota + u0) % 8 for u0 in range(0, EMB, LANES)]

        def transpose(b, ob):
            gbuf, obuf = gbufs[b], obufs[ob]

            @pl.loop(0, TB, step=LANES)
            def _(t0):
                for dt in range(LANES):
                    t = t0 + dt
                    tv = jnp.full((LANES,), t, dtype=jnp.int32)
                    for ui, u0 in enumerate(range(0, EMB, LANES)):
                        v = gbuf[t, pl.ds(u0, LANES)]
                        plsc.store_scatter(obuf, [Rv[ui], rv[ui], tv], v)

        def start_write(j, ob):
            pltpu.async_copy(out_hbm.at[j, :, wid], obufs[ob], wsems[ob])

        def wait_write(j, ob):
            pltpu.make_async_copy(
                out_hbm.at[j, :, wid], obufs[ob], wsems[ob]).wait()

        for j in range(NBUF):
            build_hidx(j, j)
            start_gather(j)

        @pl.loop(0, NJ, step=NBUF)
        def _(j0):
            for k in range(NBUF):
                j = j0 + k
                ob = k % NOB
                wait_gather(k)

                @pl.when(j >= NOB)
                def _():
                    wait_write(j - NOB, ob)

                transpose(k, ob)
                start_write(j, ob)

                @pl.when(j + NBUF < NJ)
                def _():
                    build_hidx(j + NBUF, k)
                    start_gather(k)

        for k in range(NOB):
            wait_write(NJ - NOB + k, (NJ - NOB + k) % NOB)

    return pl.kernel(
        body,
        mesh=mesh,
        compiler_params=pltpu.CompilerParams(
            use_tc_tiling_on_sc=False, needs_layout_passes=False),
        out_type=jax.ShapeDtypeStruct((NJ, EMB // 8, NW, 8, TB), jnp.float32),
        scratch_types=(
            [pltpu.VMEM((TB, NJ), jnp.int32),
             pltpu.VMEM((NBUF, TB), jnp.int32)]
            + [pltpu.VMEM((TB, EMB), jnp.float32)] * NBUF
            + [pltpu.VMEM((EMB // 8, 8, TB), jnp.float32)] * NOB
            + [pltpu.SemaphoreType.DMA] * (NBUF + NOB)
        ),
    )


def kernel(token_ids, weight):
    ni, nj = token_ids.shape
    flat = token_ids.reshape(-1).astype(jnp.int32)
    idx3 = flat.reshape(NW, TB, NJ)
    x = _build_gather()(idx3, weight)
    out = x.transpose(2, 4, 0, 1, 3).reshape(ni, nj, EMB)
    return out


# speedup vs baseline: 1.2237x; 1.2237x over previous
import functools

import jax
import jax.numpy as jnp
from jax import lax
from jax.experimental import pallas as pl
from jax.experimental.pallas import tpu as pltpu
from jax.experimental.pallas import tpu_sc as plsc

EMB = 64
NC, NS = 2, 16
NW = NC * NS
TB = 128
NJ = 200
NBUF = 8
NOB = 2
LANES = 16


@functools.lru_cache(maxsize=None)
def _build_gather():
    mesh = plsc.VectorSubcoreMesh(core_axis_name="c", subcore_axis_name="s")

    def body(idx_hbm, table_hbm, out_hbm, idxblk, hidx, *scratch):
        gbufs = scratch[:NBUF]
        obufs = scratch[NBUF:NBUF + NOB]
        gsems = scratch[NBUF + NOB:2 * NBUF + NOB]
        wsems = scratch[2 * NBUF + NOB:]
        wid = lax.axis_index("s") * NC + lax.axis_index("c")

        pltpu.sync_copy(idx_hbm.at[wid], idxblk)

        iota = lax.iota(jnp.int32, LANES)
        cvecs = [iota + LANES * cb for cb in range(TB // LANES)]

        def build_hidx(j, b):
            jv = jnp.full((LANES,), j, dtype=jnp.int32)
            for cb in range(TB // LANES):
                v = plsc.load_gather(idxblk, [cvecs[cb], jv])
                hidx[b, pl.ds(LANES * cb, LANES)] = v

        def start_gather(b):
            pltpu.async_copy(table_hbm.at[hidx.at[b]], gbufs[b], gsems[b])

        def wait_gather(b):
            pltpu.make_async_copy(
                table_hbm.at[hidx.at[b]], gbufs[b], gsems[b]).wait()

        Rv = [(iota + u0) // 8 for u0 in range(0, EMB, LANES)]
        rv = [(iota + u0) % 8 for u0 in range(0, EMB, LANES)]

        def transpose(b, ob):
            gbuf, obuf = gbufs[b], obufs[ob]

            @plsc.parallel_loop(0, TB, unroll=8)
            def _(t):
                tv = jnp.full((LANES,), t, dtype=jnp.int32)
                for ui, u0 in enumerate(range(0, EMB, LANES)):
                    v = gbuf[t, pl.ds(u0, LANES)]
                    plsc.store_scatter(obuf, [Rv[ui], rv[ui], tv], v)

        def start_write(j, ob):
            pltpu.async_copy(out_hbm.at[j, :, wid], obufs[ob], wsems[ob])

        def wait_write(j, ob):
            pltpu.make_async_copy(
                out_hbm.at[j, :, wid], obufs[ob], wsems[ob]).wait()

        for j in range(NBUF):
            build_hidx(j, j)
            start_gather(j)

        @pl.loop(0, NJ, step=NBUF)
        def _(j0):
            for k in range(NBUF):
                j = j0 + k
                ob = k % NOB
                wait_gather(k)

                @pl.when(j >= NOB)
                def _():
                    wait_write(j - NOB, ob)

                transpose(k, ob)
                start_write(j, ob)

                @pl.when(j + NBUF < NJ)
                def _():
                    build_hidx(j + NBUF, k)
                    start_gather(k)

        for k in range(NOB):
            wait_write(NJ - NOB + k, (NJ - NOB + k) % NOB)

    return pl.kernel(
        body,
        mesh=mesh,
        compiler_params=pltpu.CompilerParams(
            use_tc_tiling_on_sc=False, needs_layout_passes=False),
        out_type=jax.ShapeDtypeStruct((NJ, EMB // 8, NW, 8, TB), jnp.float32),
        scratch_types=(
            [pltpu.VMEM((TB, NJ), jnp.int32),
             pltpu.VMEM((NBUF, TB), jnp.int32)]
            + [pltpu.VMEM((TB, EMB), jnp.float32)] * NBUF
            + [pltpu.VMEM((EMB // 8, 8, TB), jnp.float32)] * NOB
            + [pltpu.SemaphoreType.DMA] * (NBUF + NOB)
        ),
    )


def kernel(token_ids, weight):
    ni, nj = token_ids.shape
    flat = token_ids.reshape(-1).astype(jnp.int32)
    idx3 = flat.reshape(NW, TB, NJ)
    x = _build_gather()(idx3, weight)
    out = x.transpose(2, 4, 0, 1, 3).reshape(ni, nj, EMB)
    return out
